# baseline (speedup 1.0000x reference)
import jax
import jax.numpy as jnp
from jax.experimental import pallas as pl

_J = 1024
_NT = 2
_CCOST = 0.25


def _vq_body(x_ref, cb_ref, q_ref, idx_ref, loss_ref):
    b = pl.program_id(0)
    i = pl.program_id(1)
    x = x_ref[0]
    cb = cb_ref[...]

    s = jax.lax.dot_general(cb, x, (((1,), (0,)), ((), ())),
                            preferred_element_type=jnp.float32)
    cnorm = jnp.sum(cb * cb, axis=1)
    xnorm = jnp.sum(x * x, axis=0)
    d = (xnorm[None, :] + cnorm[:, None]) - 2.0 * s

    minval = jnp.min(d, axis=0)
    iota = jax.lax.broadcasted_iota(jnp.int32, d.shape, 0)
    idx = jnp.min(jnp.where(d == minval[None, :], iota, _J), axis=0)

    onehot = (iota == idx[None, :]).astype(jnp.float32)
    q = jax.lax.dot_general(cb, onehot, (((0,), (0,)), ((), ())),
                            preferred_element_type=jnp.float32)

    diff = q - x
    q_ref[0] = x + diff
    idx_ref[0, 0] = idx
    part = jnp.sum(diff * diff).reshape(1, 1)

    @pl.when((b == 0) & (i == 0))
    def _init():
        loss_ref[...] = part

    @pl.when((b != 0) | (i != 0))
    def _acc():
        loss_ref[...] = loss_ref[...] + part


def kernel(x, codebook):
    B, C, H, W = x.shape
    T = H * W
    t = T // _NT
    xr = x.reshape(B, C, T)

    q, idx, loss_sum = pl.pallas_call(
        _vq_body,
        grid=(B, _NT),
        in_specs=[
            pl.BlockSpec((1, C, t), lambda b, i: (b, 0, i)),
            pl.BlockSpec((_J, C), lambda b, i: (0, 0)),
        ],
        out_specs=[
            pl.BlockSpec((1, C, t), lambda b, i: (b, 0, i)),
            pl.BlockSpec((1, 1, t), lambda b, i: (b, 0, i)),
            pl.BlockSpec((1, 1), lambda b, i: (0, 0)),
        ],
        out_shape=[
            jax.ShapeDtypeStruct((B, C, T), jnp.float32),
            jax.ShapeDtypeStruct((B, 1, T), jnp.int32),
            jax.ShapeDtypeStruct((1, 1), jnp.float32),
        ],
    )(xr, codebook)

    quantized_ste = q.reshape(B, C, H, W)
    encoding_indices = idx.reshape(B * T)
    loss = loss_sum[0, 0] * ((1.0 + _CCOST) / x.size)
    return (quantized_ste, loss, encoding_indices)


# speedup vs baseline: 1.0978x; 1.0978x over previous
import jax
import jax.numpy as jnp
from jax.experimental import pallas as pl

_J = 1024
_NT = 1
_CCOST = 0.25


def _vq_body(x_ref, cb_ref, q_ref, idx_ref, loss_ref):
    b = pl.program_id(0)
    i = pl.program_id(1)
    x = x_ref[0]
    cb = cb_ref[...]

    s = jax.lax.dot_general(cb, x, (((1,), (0,)), ((), ())),
                            preferred_element_type=jnp.float32)
    cnorm = jnp.sum(cb * cb, axis=1)
    xnorm = jnp.sum(x * x, axis=0)
    d = (xnorm[None, :] + cnorm[:, None]) - 2.0 * s

    minval = jnp.min(d, axis=0)
    iota = jax.lax.broadcasted_iota(jnp.int32, d.shape, 0)
    idx = jnp.min(jnp.where(d == minval[None, :], iota, _J), axis=0)

    onehot = (iota == idx[None, :]).astype(jnp.float32)
    q = jax.lax.dot_general(cb, onehot, (((0,), (0,)), ((), ())),
                            preferred_element_type=jnp.float32)

    diff = q - x
    q_ref[0] = x + diff
    idx_ref[0, 0] = idx
    part = jnp.sum(diff * diff).reshape(1, 1)

    @pl.when((b == 0) & (i == 0))
    def _init():
        loss_ref[...] = part

    @pl.when((b != 0) | (i != 0))
    def _acc():
        loss_ref[...] = loss_ref[...] + part


def kernel(x, codebook):
    B, C, H, W = x.shape
    T = H * W
    t = T // _NT
    xr = x.reshape(B, C, T)

    q, idx, loss_sum = pl.pallas_call(
        _vq_body,
        grid=(B, _NT),
        in_specs=[
            pl.BlockSpec((1, C, t), lambda b, i: (b, 0, i)),
            pl.BlockSpec((_J, C), lambda b, i: (0, 0)),
        ],
        out_specs=[
            pl.BlockSpec((1, C, t), lambda b, i: (b, 0, i)),
            pl.BlockSpec((1, 1, t), lambda b, i: (b, 0, i)),
            pl.BlockSpec((1, 1), lambda b, i: (0, 0)),
        ],
        out_shape=[
            jax.ShapeDtypeStruct((B, C, T), jnp.float32),
            jax.ShapeDtypeStruct((B, 1, T), jnp.int32),
            jax.ShapeDtypeStruct((1, 1), jnp.float32),
        ],
    )(xr, codebook)

    quantized_ste = q.reshape(B, C, H, W)
    encoding_indices = idx.reshape(B * T)
    loss = loss_sum[0, 0] * ((1.0 + _CCOST) / x.size)
    return (quantized_ste, loss, encoding_indices)


# speedup vs baseline: 1.1166x; 1.0171x over previous
import jax
import jax.numpy as jnp
from jax.experimental import pallas as pl

_J = 1024
_NT = 1
_CCOST = 0.25


def _vq_body(x_ref, cb_ref, q_ref, idx_ref, loss_ref):
    b = pl.program_id(0)
    i = pl.program_id(1)
    x = x_ref[0]
    cb = cb_ref[...]

    cbn = -(cb + cb)
    sn = jax.lax.dot_general(cbn, x, (((1,), (0,)), ((), ())),
                             preferred_element_type=jnp.float32)
    cnorm = jnp.sum(cb * cb, axis=1)
    xnorm = jnp.sum(x * x, axis=0)
    d = (xnorm[None, :] + cnorm[:, None]) + sn

    minval = jnp.min(d, axis=0)
    iota = jax.lax.broadcasted_iota(jnp.int32, d.shape, 0)
    idx = jnp.min(jnp.where(d == minval[None, :], iota, _J), axis=0)

    onehot = (iota == idx[None, :]).astype(jnp.float32)
    q = jax.lax.dot_general(cb, onehot, (((0,), (0,)), ((), ())),
                            preferred_element_type=jnp.float32)

    diff = q - x
    q_ref[0] = x + diff
    idx_ref[0, 0] = idx
    part = jnp.sum(diff * diff).reshape(1, 1)

    @pl.when((b == 0) & (i == 0))
    def _init():
        loss_ref[...] = part

    @pl.when((b != 0) | (i != 0))
    def _acc():
        loss_ref[...] = loss_ref[...] + part


def kernel(x, codebook):
    B, C, H, W = x.shape
    T = H * W
    t = T // _NT
    xr = x.reshape(B, C, T)

    q, idx, loss_sum = pl.pallas_call(
        _vq_body,
        grid=(B, _NT),
        in_specs=[
            pl.BlockSpec((1, C, t), lambda b, i: (b, 0, i)),
            pl.BlockSpec((_J, C), lambda b, i: (0, 0)),
        ],
        out_specs=[
            pl.BlockSpec((1, C, t), lambda b, i: (b, 0, i)),
            pl.BlockSpec((1, 1, t), lambda b, i: (b, 0, i)),
            pl.BlockSpec((1, 1), lambda b, i: (0, 0)),
        ],
        out_shape=[
            jax.ShapeDtypeStruct((B, C, T), jnp.float32),
            jax.ShapeDtypeStruct((B, 1, T), jnp.int32),
            jax.ShapeDtypeStruct((1, 1), jnp.float32),
        ],
    )(xr, codebook)

    quantized_ste = q.reshape(B, C, H, W)
    encoding_indices = idx.reshape(B * T)
    loss = loss_sum[0, 0] * ((1.0 + _CCOST) / x.size)
    return (quantized_ste, loss, encoding_indices)


# speedup vs baseline: 1.2110x; 1.0846x over previous
import jax
import jax.numpy as jnp
from jax.experimental import pallas as pl

_J = 1024
_NB = 2
_R = 8
_CCOST = 0.25


def _vq_body(x_ref, cb_ref, q_ref, idx_ref, loss_ref):
    b = pl.program_id(0)
    cb = cb_ref[...]
    for k in range(x_ref.shape[0]):
        _vq_one(k, b, x_ref, cb, q_ref, idx_ref, loss_ref)


def _vq_one(k, b, x_ref, cb, q_ref, idx_ref, loss_ref):
    x = x_ref[k]
    t = x.shape[1]

    cbn = -(cb + cb)
    sn = jax.lax.dot_general(cbn, x, (((1,), (0,)), ((), ())),
                             preferred_element_type=jnp.float32)
    cnorm = jnp.sum(cb * cb, axis=1)
    xnorm = jnp.sum(x * x, axis=0)

    iota_r = jax.lax.broadcasted_iota(jnp.int32, (_R, t), 0)
    minv = jnp.full((_R, t), jnp.inf, jnp.float32)
    mini = jnp.zeros((_R, t), jnp.int32)
    for c in range(_J // _R):
        r0 = c * _R
        d_chunk = (xnorm[None, :] + cnorm[r0:r0 + _R, None]) + sn[r0:r0 + _R]
        upd = d_chunk < minv
        minv = jnp.where(upd, d_chunk, minv)
        mini = jnp.where(upd, iota_r + r0, mini)
    span = _R
    while span > 1:
        span //= 2
        v2 = minv[span:2 * span]
        i2 = mini[span:2 * span]
        v1 = minv[:span]
        i1 = mini[:span]
        lt = (v2 < v1) | ((v2 == v1) & (i2 < i1))
        minv = jnp.where(lt, v2, v1)
        mini = jnp.where(lt, i2, i1)
    idx = mini[0]

    iota = jax.lax.broadcasted_iota(jnp.int32, (_J, t), 0)
    onehot = (iota == idx[None, :]).astype(jnp.float32)
    q = jax.lax.dot_general(cb, onehot, (((0,), (0,)), ((), ())),
                            preferred_element_type=jnp.float32)

    diff = q - x
    q_ref[k] = x + diff
    idx_ref[k, 0] = idx
    part = jnp.sum(diff * diff).reshape(1, 1)

    if k == 0:
        @pl.when(b == 0)
        def _init():
            loss_ref[...] = part

        @pl.when(b != 0)
        def _acc():
            loss_ref[...] = loss_ref[...] + part
    else:
        loss_ref[...] = loss_ref[...] + part


def kernel(x, codebook):
    B, C, H, W = x.shape
    T = H * W
    nb = B // _NB
    xr = x.reshape(B, C, T)

    q, idx, loss_sum = pl.pallas_call(
        _vq_body,
        grid=(nb,),
        in_specs=[
            pl.BlockSpec((_NB, C, T), lambda b: (b, 0, 0)),
            pl.BlockSpec((_J, C), lambda b: (0, 0)),
        ],
        out_specs=[
            pl.BlockSpec((_NB, C, T), lambda b: (b, 0, 0)),
            pl.BlockSpec((_NB, 1, T), lambda b: (b, 0, 0)),
            pl.BlockSpec((1, 1), lambda b: (0, 0)),
        ],
        out_shape=[
            jax.ShapeDtypeStruct((B, C, T), jnp.float32),
            jax.ShapeDtypeStruct((B, 1, T), jnp.int32),
            jax.ShapeDtypeStruct((1, 1), jnp.float32),
        ],
    )(xr, codebook)

    quantized_ste = q.reshape(B, C, H, W)
    encoding_indices = idx.reshape(B * T)
    loss = loss_sum[0, 0] * ((1.0 + _CCOST) / x.size)
    return (quantized_ste, loss, encoding_indices)


# speedup vs baseline: 1.2124x; 1.0011x over previous
import jax
import jax.numpy as jnp
from jax.experimental import pallas as pl

_J = 1024
_NB = 2
_R = 8
_CCOST = 0.25


def _vq_body(x_ref, cb_ref, q_ref, idx_ref, loss_ref):
    b = pl.program_id(0)
    cb = cb_ref[...]
    cbn = -(cb + cb)
    cnorm = jnp.sum(cb * cb, axis=1)
    for k in range(x_ref.shape[0]):
        _vq_one(k, b, x_ref, cb, cbn, cnorm, q_ref, idx_ref, loss_ref)


def _vq_one(k, b, x_ref, cb, cbn, cnorm, q_ref, idx_ref, loss_ref):
    x = x_ref[k]
    t = x.shape[1]

    sn = jax.lax.dot_general(cbn, x, (((1,), (0,)), ((), ())),
                             preferred_element_type=jnp.float32)
    xnorm = jnp.sum(x * x, axis=0)

    iota_r = jax.lax.broadcasted_iota(jnp.int32, (_R, t), 0)
    minv = jnp.full((_R, t), jnp.inf, jnp.float32)
    mini = jnp.zeros((_R, t), jnp.int32)
    for c in range(_J // _R):
        r0 = c * _R
        d_chunk = (xnorm[None, :] + cnorm[r0:r0 + _R, None]) + sn[r0:r0 + _R]
        upd = d_chunk < minv
        minv = jnp.where(upd, d_chunk, minv)
        mini = jnp.where(upd, iota_r + r0, mini)
    span = _R
    while span > 1:
        span //= 2
        v2 = minv[span:2 * span]
        i2 = mini[span:2 * span]
        v1 = minv[:span]
        i1 = mini[:span]
        lt = (v2 < v1) | ((v2 == v1) & (i2 < i1))
        minv = jnp.where(lt, v2, v1)
        mini = jnp.where(lt, i2, i1)
    idx = mini[0]

    iota = jax.lax.broadcasted_iota(jnp.int32, (_J, t), 0)
    onehot = (iota == idx[None, :]).astype(jnp.float32)
    q = jax.lax.dot_general(cb, onehot, (((0,), (0,)), ((), ())),
                            preferred_element_type=jnp.float32)

    diff = q - x
    q_ref[k] = x + diff
    idx_ref[k, 0] = idx
    part = jnp.sum(diff * diff).reshape(1, 1)

    if k == 0:
        @pl.when(b == 0)
        def _init():
            loss_ref[...] = part

        @pl.when(b != 0)
        def _acc():
            loss_ref[...] = loss_ref[...] + part
    else:
        loss_ref[...] = loss_ref[...] + part


def kernel(x, codebook):
    B, C, H, W = x.shape
    T = H * W
    nb = B // _NB
    xr = x.reshape(B, C, T)

    q, idx, loss_sum = pl.pallas_call(
        _vq_body,
        grid=(nb,),
        in_specs=[
            pl.BlockSpec((_NB, C, T), lambda b: (b, 0, 0)),
            pl.BlockSpec((_J, C), lambda b: (0, 0)),
        ],
        out_specs=[
            pl.BlockSpec((_NB, C, T), lambda b: (b, 0, 0)),
            pl.BlockSpec((_NB, 1, T), lambda b: (b, 0, 0)),
            pl.BlockSpec((1, 1), lambda b: (0, 0)),
        ],
        out_shape=[
            jax.ShapeDtypeStruct((B, C, T), jnp.float32),
            jax.ShapeDtypeStruct((B, 1, T), jnp.int32),
            jax.ShapeDtypeStruct((1, 1), jnp.float32),
        ],
    )(xr, codebook)

    quantized_ste = q.reshape(B, C, H, W)
    encoding_indices = idx.reshape(B * T)
    loss = loss_sum[0, 0] * ((1.0 + _CCOST) / x.size)
    return (quantized_ste, loss, encoding_indices)
